# baseline (speedup 1.0000x reference)
import functools

import jax
import jax.numpy as jnp
from jax import lax
from jax.experimental import pallas as pl
from jax.experimental.pallas import tpu as pltpu
from jax.experimental.pallas import tpu_sc as plsc

NC = 2
NS = 16
L = 16
NW = NC * NS

TILE = 256
LOG_TILE = 8


def _gate_kernel(x_ref, gwt_ref, i1_ref, i2_ref, w1_ref, w2_ref):
    x = x_ref[...]
    logits = jax.lax.dot_general(
        x, gwt_ref[...], (((1,), (0,)), ((), ())),
        preferred_element_type=jnp.float32)
    t, e = logits.shape
    eidx = jax.lax.broadcasted_iota(jnp.int32, (t, e), 1)
    m1 = jnp.max(logits, axis=1, keepdims=True)
    i1 = jnp.min(jnp.where(logits == m1, eidx, e), axis=1, keepdims=True)
    lm = jnp.where(eidx == i1, -1e30, logits)
    m2 = jnp.max(lm, axis=1, keepdims=True)
    i2 = jnp.min(jnp.where(lm == m2, eidx, e), axis=1, keepdims=True)
    w1 = 1.0 / (1.0 + jnp.exp(m2 - m1))
    i1_ref[...] = i1
    i2_ref[...] = i2
    w1_ref[...] = w1
    w2_ref[...] = 1.0 - w1


def _rgather(v, idx):
    return v.at[idx].get(mode="promise_in_bounds")


def _prefix_same(ev, lanes):
    acc = jnp.zeros((L,), jnp.int32)
    for sh in range(1, L):
        shifted = _rgather(ev, jnp.maximum(lanes - sh, 0))
        acc = acc + jnp.where((lanes >= sh) & (shifted == ev), 1, 0)
    return acc


def _hist(ev, lanes):
    h = jnp.zeros((L,), jnp.int32)
    for l in range(L):
        h = h + jnp.where(lanes == ev[l], 1, 0)
    return h


def _excl_prefix_sum(v, lanes):
    acc = jnp.zeros((L,), jnp.int32)
    for sh in range(1, L):
        acc = acc + jnp.where(lanes >= sh, _rgather(v, jnp.maximum(lanes - sh, 0)), 0)
    return acc


def _route_body(i1_hbm, i2_hbm, x_hbm, pose_o, poso_o, xs_o, texp_o,
                tvalid_o, hist_o,
                i1l, i2l, pel, pol, hv, allh, texpv, tvalv,
                idx2a, idx2b, xloc0, xloc1, seml0, seml1, sems, *,
                n_exp, toks, maxt_pad, ch):
    cid = lax.axis_index("c")
    sid = lax.axis_index("s")
    tpr = toks // NS
    nv = tpr // L
    t0 = sid * tpr
    pltpu.sync_copy(i1_hbm.at[pl.ds(t0, tpr)], i1l)
    pltpu.sync_copy(i2_hbm.at[pl.ds(t0, tpr)], i2l)
    lanes = lax.iota(jnp.int32, L)

    cnte = jnp.zeros((L,), jnp.int32)
    for m in range(nv):
        sl = pl.ds(m * L, L)
        i1v = i1l[sl]
        pel[sl] = _rgather(cnte, i1v) + _prefix_same(i1v, lanes)
        cnte = cnte + _hist(i1v, lanes)
    cnto = jnp.zeros((L,), jnp.int32)
    for m in range(nv):
        sl = pl.ds(m * L, L)
        i2v = i2l[sl]
        pol[sl] = (_rgather(cnte, i2v) + _rgather(cnto, i2v)
                   + _prefix_same(i2v, lanes))
        cnto = cnto + _hist(i2v, lanes)

    hv[...] = cnte + cnto
    pltpu.sync_copy(hv, hist_o.at[sid])
    plsc.subcore_barrier()
    pltpu.sync_copy(hist_o, allh)
    total = jnp.zeros((L,), jnp.int32)
    mybase = jnp.zeros((L,), jnp.int32)
    for sp in range(NS):
        row = allh[sp]
        total = total + row
        mybase = mybase + jnp.where(sp < sid, row, 0)
    pt = ((total + (TILE - 1)) >> LOG_TILE) << LOG_TILE
    padoff = _excl_prefix_sum(pt, lanes)
    base_add = padoff + mybase

    for m in range(nv):
        sl = pl.ds(m * L, L)
        pel[sl] = pel[sl] + _rgather(base_add, i1l[sl])
        pol[sl] = pol[sl] + _rgather(base_add, i2l[sl])
    pltpu.sync_copy(pel, pose_o.at[pl.ds(t0, tpr)])
    pltpu.sync_copy(pol, poso_o.at[pl.ds(t0, tpr)])

    tstart = padoff >> LOG_TILE
    nt = ((padoff + pt) >> LOG_TILE)[n_exp - 1]
    for bb in range(maxt_pad // L):
        tv = lax.iota(jnp.int32, L) + bb * L
        tx = jnp.zeros((L,), jnp.int32)
        for e in range(1, n_exp):
            tx = tx + jnp.where(tv >= tstart[e], 1, 0)
        val = jnp.where(tv < nt, 1, 0)
        texpv[pl.ds(bb * L, L)] = jnp.where(val == 1, tx, n_exp - 1)
        tvalv[pl.ds(bb * L, L)] = val
    pltpu.sync_copy(texpv, texp_o)
    pltpu.sync_copy(tvalv, tvalid_o)

    half = tpr // NC
    off = cid * half
    nch = half // ch
    bufs = [(idx2a, xloc0, seml0), (idx2b, xloc1, seml1)]
    loads = {}
    stores = {}

    def stage_idx(c):
        idx2, _, _ = bufs[c % 2]
        for v in range(ch // L):
            slv = pl.ds(off + c * ch + v * L, L)
            idx2[0, pl.ds(v * L, L)] = pel[slv]
            idx2[1, pl.ds(v * L, L)] = pol[slv]

    def issue_loads(c):
        _, xloc, seml = bufs[c % 2]
        tc = t0 + off + c * ch
        loads[c] = pltpu.async_copy(x_hbm.at[pl.ds(tc, ch)], xloc, seml)

    issue_loads(0)
    if nch > 1:
        issue_loads(1)
    for c in range(nch):
        idx2, xloc, _ = bufs[c % 2]
        stage_idx(c)
        loads[c].wait()
        stores[c] = [
            pltpu.async_copy(xloc, xs_o.at[idx2.at[0]], sems),
            pltpu.async_copy(xloc, xs_o.at[idx2.at[1]], sems),
        ]
        if c + 2 < nch:
            for cp in stores[c]:
                cp.wait()
            stores[c] = []
            issue_loads(c + 2)
    for c in range(nch):
        for cp in stores.get(c, []):
            cp.wait()


def _combine_body(pose_hbm, poso_hbm, w1_hbm, w2_hbm, y_hbm, out_hbm,
                  pel, pol, w1l, w2l, bufa0, bufb0, bufa1, bufb1,
                  sem0, sem1, *, tpw, h):
    cid = lax.axis_index("c")
    sid = lax.axis_index("s")
    wid = sid * NC + cid
    t0 = wid * tpw
    pltpu.sync_copy(pose_hbm.at[pl.ds(t0, tpw)], pel)
    pltpu.sync_copy(poso_hbm.at[pl.ds(t0, tpw)], pol)
    pltpu.sync_copy(w1_hbm.at[pl.ds(t0, tpw)], w1l)
    pltpu.sync_copy(w2_hbm.at[pl.ds(t0, tpw)], w2l)
    bufs = [(bufa0, bufb0, sem0), (bufa1, bufb1, sem1)]
    nch = tpw // L

    def start(c):
        sl = pl.ds(c * L, L)
        ba, bb, sem = bufs[c % 2]
        return (pltpu.async_copy(y_hbm.at[pel.at[sl]], ba, sem),
                pltpu.async_copy(y_hbm.at[pol.at[sl]], bb, sem))

    pending = start(0)
    for c in range(nch):
        cpa, cpb = pending
        cpa.wait()
        cpb.wait()
        if c + 1 < nch:
            pending = start(c + 1)
        ba, bb, _ = bufs[c % 2]
        sl = pl.ds(c * L, L)
        w1v = w1l[sl]
        w2v = w2l[sl]

        def tbody(t, _, ba=ba, bb=bb, w1v=w1v, w2v=w2v):
            wsa = _rgather(w1v, jnp.broadcast_to(t, (L,)))
            wsb = _rgather(w2v, jnp.broadcast_to(t, (L,)))
            for k in range(h // L):
                ksl = pl.ds(k * L, L)
                ba[t, ksl] = ba[t, ksl] * wsa + bb[t, ksl] * wsb
            return 0

        lax.fori_loop(0, L, tbody, 0)
        pltpu.sync_copy(ba, out_hbm.at[pl.ds(t0 + c * L, L)])


def _mm_kernel(texp_ref, tval_ref, xs_ref, wg_ref, wu_ref, wd_ref, y_ref):
    i = pl.program_id(0)

    @pl.when(tval_ref[i] == 1)
    def _():
        xc = xs_ref[...].astype(jnp.bfloat16)
        g = jax.lax.dot_general(xc, wg_ref[0], (((1,), (0,)), ((), ())),
                                preferred_element_type=jnp.float32)
        u = jax.lax.dot_general(xc, wu_ref[0], (((1,), (0,)), ((), ())),
                                preferred_element_type=jnp.float32)
        hh = (g * jax.lax.logistic(g) * u).astype(jnp.bfloat16)
        y_ref[...] = jax.lax.dot_general(hh, wd_ref[0], (((1,), (0,)), ((), ())),
                                         preferred_element_type=jnp.float32)


def kernel(hidden_states, gate_w, w_gate, w_up, w_down):
    b, s, h = hidden_states.shape
    e_num, _, f = w_gate.shape
    t = b * s
    maxr = 2 * t + e_num * TILE
    maxt = maxr // TILE
    maxt_pad = ((maxt + L - 1) // L) * L
    tpw = t // NW

    x = hidden_states.reshape(t, h)
    i1, i2, w1, w2 = pl.pallas_call(
        _gate_kernel,
        out_shape=[
            jax.ShapeDtypeStruct((t, 1), jnp.int32),
            jax.ShapeDtypeStruct((t, 1), jnp.int32),
            jax.ShapeDtypeStruct((t, 1), jnp.float32),
            jax.ShapeDtypeStruct((t, 1), jnp.float32),
        ],
    )(x, gate_w.T)
    i1f = i1.reshape(t)
    i2f = i2.reshape(t)
    w1f = w1.reshape(t)
    w2f = w2.reshape(t)

    mesh = plsc.VectorSubcoreMesh(core_axis_name="c", subcore_axis_name="s",
                                  num_cores=NC, num_subcores=NS)
    tpr = t // NS

    ch = 32
    route = pl.kernel(
        functools.partial(_route_body, n_exp=e_num, toks=t,
                          maxt_pad=maxt_pad, ch=ch),
        out_type=[
            jax.ShapeDtypeStruct((t,), jnp.int32),
            jax.ShapeDtypeStruct((t,), jnp.int32),
            jax.ShapeDtypeStruct((maxr, h), jnp.float32),
            jax.ShapeDtypeStruct((maxt_pad,), jnp.int32),
            jax.ShapeDtypeStruct((maxt_pad,), jnp.int32),
            jax.ShapeDtypeStruct((NS, L), jnp.int32),
        ],
        mesh=mesh,
        scratch_types=[
            pltpu.VMEM((tpr,), jnp.int32),
            pltpu.VMEM((tpr,), jnp.int32),
            pltpu.VMEM((tpr,), jnp.int32),
            pltpu.VMEM((tpr,), jnp.int32),
            pltpu.VMEM((L,), jnp.int32),
            pltpu.VMEM((NS, L), jnp.int32),
            pltpu.VMEM((maxt_pad,), jnp.int32),
            pltpu.VMEM((maxt_pad,), jnp.int32),
            pltpu.VMEM((2, ch), jnp.int32),
            pltpu.VMEM((2, ch), jnp.int32),
            pltpu.VMEM((ch, h), jnp.float32),
            pltpu.VMEM((ch, h), jnp.float32),
            pltpu.SemaphoreType.DMA,
            pltpu.SemaphoreType.DMA,
            pltpu.SemaphoreType.DMA,
        ],
    )
    pose, poso, xs, texp, tvalid, _ = route(i1f, i2f, x)


    grid_spec = pltpu.PrefetchScalarGridSpec(
        num_scalar_prefetch=2,
        grid=(maxt,),
        in_specs=[
            pl.BlockSpec((TILE, h), lambda i, texp, tval: (i, 0)),
            pl.BlockSpec((1, h, f), lambda i, texp, tval: (texp[i], 0, 0)),
            pl.BlockSpec((1, h, f), lambda i, texp, tval: (texp[i], 0, 0)),
            pl.BlockSpec((1, f, h), lambda i, texp, tval: (texp[i], 0, 0)),
        ],
        out_specs=pl.BlockSpec((TILE, h), lambda i, texp, tval: (i, 0)),
    )
    y = pl.pallas_call(
        _mm_kernel,
        grid_spec=grid_spec,
        out_shape=jax.ShapeDtypeStruct((maxr, h), jnp.float32),
    )(texp, tvalid, xs,
      w_gate.astype(jnp.bfloat16),
      w_up.astype(jnp.bfloat16),
      w_down.astype(jnp.bfloat16))

    combine = pl.kernel(
        functools.partial(_combine_body, tpw=tpw, h=h),
        out_type=jax.ShapeDtypeStruct((t, h), jnp.float32),
        mesh=mesh,
        scratch_types=[
            pltpu.VMEM((tpw,), jnp.int32),
            pltpu.VMEM((tpw,), jnp.int32),
            pltpu.VMEM((tpw,), jnp.float32),
            pltpu.VMEM((tpw,), jnp.float32),
            pltpu.VMEM((L, h), jnp.float32),
            pltpu.VMEM((L, h), jnp.float32),
            pltpu.VMEM((L, h), jnp.float32),
            pltpu.VMEM((L, h), jnp.float32),
            pltpu.SemaphoreType.DMA,
            pltpu.SemaphoreType.DMA,
        ],
    )
    out = combine(pose, poso, w1f, w2f, y)
    return out.reshape(b, s, h)


# speedup vs baseline: 1.0228x; 1.0228x over previous
import functools

import jax
import jax.numpy as jnp
from jax import lax
from jax.experimental import pallas as pl
from jax.experimental.pallas import tpu as pltpu
from jax.experimental.pallas import tpu_sc as plsc

NC = 2
NS = 16
L = 16
NW = NC * NS

TILE = 512
LOG_TILE = 9


def _gate_kernel(x_ref, gwt_ref, i1_ref, i2_ref, w1_ref, w2_ref):
    x = x_ref[...]
    logits = jax.lax.dot_general(
        x, gwt_ref[...], (((1,), (0,)), ((), ())),
        preferred_element_type=jnp.float32)
    t, e = logits.shape
    eidx = jax.lax.broadcasted_iota(jnp.int32, (t, e), 1)
    m1 = jnp.max(logits, axis=1, keepdims=True)
    i1 = jnp.min(jnp.where(logits == m1, eidx, e), axis=1, keepdims=True)
    lm = jnp.where(eidx == i1, -1e30, logits)
    m2 = jnp.max(lm, axis=1, keepdims=True)
    i2 = jnp.min(jnp.where(lm == m2, eidx, e), axis=1, keepdims=True)
    w1 = 1.0 / (1.0 + jnp.exp(m2 - m1))
    i1_ref[...] = i1
    i2_ref[...] = i2
    w1_ref[...] = w1
    w2_ref[...] = 1.0 - w1


def _rgather(v, idx):
    return v.at[idx].get(mode="promise_in_bounds")


def _prefix_same(ev, lanes):
    acc = jnp.zeros((L,), jnp.int32)
    for sh in range(1, L):
        shifted = _rgather(ev, jnp.maximum(lanes - sh, 0))
        acc = acc + jnp.where((lanes >= sh) & (shifted == ev), 1, 0)
    return acc


def _hist(ev, lanes):
    h = jnp.zeros((L,), jnp.int32)
    for l in range(L):
        h = h + jnp.where(lanes == ev[l], 1, 0)
    return h


def _excl_prefix_sum(v, lanes):
    acc = jnp.zeros((L,), jnp.int32)
    for sh in range(1, L):
        acc = acc + jnp.where(lanes >= sh, _rgather(v, jnp.maximum(lanes - sh, 0)), 0)
    return acc


def _route_body(i1_hbm, i2_hbm, x_hbm, pose_o, poso_o, xs_o, texp_o,
                tvalid_o, hist_o,
                i1l, i2l, pel, pol, hv, allh, texpv, tvalv,
                idx2a, idx2b, xloc0, xloc1, seml0, seml1, sems, *,
                n_exp, toks, maxt_pad, ch):
    cid = lax.axis_index("c")
    sid = lax.axis_index("s")
    tpr = toks // NS
    nv = tpr // L
    t0 = sid * tpr
    pltpu.sync_copy(i1_hbm.at[pl.ds(t0, tpr)], i1l)
    pltpu.sync_copy(i2_hbm.at[pl.ds(t0, tpr)], i2l)
    lanes = lax.iota(jnp.int32, L)

    cnte = jnp.zeros((L,), jnp.int32)
    for m in range(nv):
        sl = pl.ds(m * L, L)
        i1v = i1l[sl]
        pel[sl] = _rgather(cnte, i1v) + _prefix_same(i1v, lanes)
        cnte = cnte + _hist(i1v, lanes)
    cnto = jnp.zeros((L,), jnp.int32)
    for m in range(nv):
        sl = pl.ds(m * L, L)
        i2v = i2l[sl]
        pol[sl] = (_rgather(cnte, i2v) + _rgather(cnto, i2v)
                   + _prefix_same(i2v, lanes))
        cnto = cnto + _hist(i2v, lanes)

    hv[...] = cnte + cnto
    pltpu.sync_copy(hv, hist_o.at[sid])
    plsc.subcore_barrier()
    pltpu.sync_copy(hist_o, allh)
    total = jnp.zeros((L,), jnp.int32)
    mybase = jnp.zeros((L,), jnp.int32)
    for sp in range(NS):
        row = allh[sp]
        total = total + row
        mybase = mybase + jnp.where(sp < sid, row, 0)
    pt = ((total + (TILE - 1)) >> LOG_TILE) << LOG_TILE
    padoff = _excl_prefix_sum(pt, lanes)
    base_add = padoff + mybase

    for m in range(nv):
        sl = pl.ds(m * L, L)
        pel[sl] = pel[sl] + _rgather(base_add, i1l[sl])
        pol[sl] = pol[sl] + _rgather(base_add, i2l[sl])
    pltpu.sync_copy(pel, pose_o.at[pl.ds(t0, tpr)])
    pltpu.sync_copy(pol, poso_o.at[pl.ds(t0, tpr)])

    tstart = padoff >> LOG_TILE
    nt = ((padoff + pt) >> LOG_TILE)[n_exp - 1]
    for bb in range(maxt_pad // L):
        tv = lax.iota(jnp.int32, L) + bb * L
        tx = jnp.zeros((L,), jnp.int32)
        for e in range(1, n_exp):
            tx = tx + jnp.where(tv >= tstart[e], 1, 0)
        val = jnp.where(tv < nt, 1, 0)
        texpv[pl.ds(bb * L, L)] = jnp.where(val == 1, tx, n_exp - 1)
        tvalv[pl.ds(bb * L, L)] = val
    pltpu.sync_copy(texpv, texp_o)
    pltpu.sync_copy(tvalv, tvalid_o)

    half = tpr // NC
    off = cid * half
    nch = half // ch
    bufs = [(idx2a, xloc0, seml0), (idx2b, xloc1, seml1)]
    loads = {}
    stores = {}

    def stage_idx(c):
        idx2, _, _ = bufs[c % 2]
        for v in range(ch // L):
            slv = pl.ds(off + c * ch + v * L, L)
            idx2[0, pl.ds(v * L, L)] = pel[slv]
            idx2[1, pl.ds(v * L, L)] = pol[slv]

    def issue_loads(c):
        _, xloc, seml = bufs[c % 2]
        tc = t0 + off + c * ch
        loads[c] = pltpu.async_copy(x_hbm.at[pl.ds(tc, ch)], xloc, seml)

    issue_loads(0)
    if nch > 1:
        issue_loads(1)
    for c in range(nch):
        idx2, xloc, _ = bufs[c % 2]
        stage_idx(c)
        loads[c].wait()
        stores[c] = [
            pltpu.async_copy(xloc, xs_o.at[idx2.at[0]], sems),
            pltpu.async_copy(xloc, xs_o.at[idx2.at[1]], sems),
        ]
        if c + 2 < nch:
            for cp in stores[c]:
                cp.wait()
            stores[c] = []
            issue_loads(c + 2)
    for c in range(nch):
        for cp in stores.get(c, []):
            cp.wait()


def _combine_body(pose_hbm, poso_hbm, w1_hbm, w2_hbm, y_hbm, out_hbm,
                  pel, pol, w1l, w2l, bufa0, bufb0, bufa1, bufb1,
                  sem0, sem1, *, tpw, h):
    cid = lax.axis_index("c")
    sid = lax.axis_index("s")
    wid = sid * NC + cid
    t0 = wid * tpw
    pltpu.sync_copy(pose_hbm.at[pl.ds(t0, tpw)], pel)
    pltpu.sync_copy(poso_hbm.at[pl.ds(t0, tpw)], pol)
    pltpu.sync_copy(w1_hbm.at[pl.ds(t0, tpw)], w1l)
    pltpu.sync_copy(w2_hbm.at[pl.ds(t0, tpw)], w2l)
    bufs = [(bufa0, bufb0, sem0), (bufa1, bufb1, sem1)]
    nch = tpw // L

    def start(c):
        sl = pl.ds(c * L, L)
        ba, bb, sem = bufs[c % 2]
        return (pltpu.async_copy(y_hbm.at[pel.at[sl]], ba, sem),
                pltpu.async_copy(y_hbm.at[pol.at[sl]], bb, sem))

    pending = start(0)
    for c in range(nch):
        cpa, cpb = pending
        cpa.wait()
        cpb.wait()
        if c + 1 < nch:
            pending = start(c + 1)
        ba, bb, _ = bufs[c % 2]
        sl = pl.ds(c * L, L)
        w1v = w1l[sl]
        w2v = w2l[sl]

        def tbody(t, _, ba=ba, bb=bb, w1v=w1v, w2v=w2v):
            wsa = _rgather(w1v, jnp.broadcast_to(t, (L,)))
            wsb = _rgather(w2v, jnp.broadcast_to(t, (L,)))
            for k in range(h // L):
                ksl = pl.ds(k * L, L)
                ba[t, ksl] = ba[t, ksl] * wsa + bb[t, ksl] * wsb
            return 0

        lax.fori_loop(0, L, tbody, 0)
        pltpu.sync_copy(ba, out_hbm.at[pl.ds(t0 + c * L, L)])


def _mm_kernel(texp_ref, tval_ref, xs_ref, wg_ref, wu_ref, wd_ref, y_ref):
    i = pl.program_id(0)

    @pl.when(tval_ref[i] == 1)
    def _():
        xc = xs_ref[...].astype(jnp.bfloat16)
        g = jax.lax.dot_general(xc, wg_ref[0], (((1,), (0,)), ((), ())),
                                preferred_element_type=jnp.float32)
        u = jax.lax.dot_general(xc, wu_ref[0], (((1,), (0,)), ((), ())),
                                preferred_element_type=jnp.float32)
        hh = (g * jax.lax.logistic(g) * u).astype(jnp.bfloat16)
        y_ref[...] = jax.lax.dot_general(hh, wd_ref[0], (((1,), (0,)), ((), ())),
                                         preferred_element_type=jnp.float32)


def kernel(hidden_states, gate_w, w_gate, w_up, w_down):
    b, s, h = hidden_states.shape
    e_num, _, f = w_gate.shape
    t = b * s
    maxr = 2 * t + e_num * TILE
    maxt = maxr // TILE
    maxt_pad = ((maxt + L - 1) // L) * L
    tpw = t // NW

    x = hidden_states.reshape(t, h)
    i1, i2, w1, w2 = pl.pallas_call(
        _gate_kernel,
        out_shape=[
            jax.ShapeDtypeStruct((t, 1), jnp.int32),
            jax.ShapeDtypeStruct((t, 1), jnp.int32),
            jax.ShapeDtypeStruct((t, 1), jnp.float32),
            jax.ShapeDtypeStruct((t, 1), jnp.float32),
        ],
    )(x, gate_w.T)
    i1f = i1.reshape(t)
    i2f = i2.reshape(t)
    w1f = w1.reshape(t)
    w2f = w2.reshape(t)

    mesh = plsc.VectorSubcoreMesh(core_axis_name="c", subcore_axis_name="s",
                                  num_cores=NC, num_subcores=NS)
    tpr = t // NS

    ch = 32
    route = pl.kernel(
        functools.partial(_route_body, n_exp=e_num, toks=t,
                          maxt_pad=maxt_pad, ch=ch),
        out_type=[
            jax.ShapeDtypeStruct((t,), jnp.int32),
            jax.ShapeDtypeStruct((t,), jnp.int32),
            jax.ShapeDtypeStruct((maxr, h), jnp.float32),
            jax.ShapeDtypeStruct((maxt_pad,), jnp.int32),
            jax.ShapeDtypeStruct((maxt_pad,), jnp.int32),
            jax.ShapeDtypeStruct((NS, L), jnp.int32),
        ],
        mesh=mesh,
        scratch_types=[
            pltpu.VMEM((tpr,), jnp.int32),
            pltpu.VMEM((tpr,), jnp.int32),
            pltpu.VMEM((tpr,), jnp.int32),
            pltpu.VMEM((tpr,), jnp.int32),
            pltpu.VMEM((L,), jnp.int32),
            pltpu.VMEM((NS, L), jnp.int32),
            pltpu.VMEM((maxt_pad,), jnp.int32),
            pltpu.VMEM((maxt_pad,), jnp.int32),
            pltpu.VMEM((2, ch), jnp.int32),
            pltpu.VMEM((2, ch), jnp.int32),
            pltpu.VMEM((ch, h), jnp.float32),
            pltpu.VMEM((ch, h), jnp.float32),
            pltpu.SemaphoreType.DMA,
            pltpu.SemaphoreType.DMA,
            pltpu.SemaphoreType.DMA,
        ],
    )
    pose, poso, xs, texp, tvalid, _ = route(i1f, i2f, x)


    grid_spec = pltpu.PrefetchScalarGridSpec(
        num_scalar_prefetch=2,
        grid=(maxt,),
        in_specs=[
            pl.BlockSpec((TILE, h), lambda i, texp, tval: (i, 0)),
            pl.BlockSpec((1, h, f), lambda i, texp, tval: (texp[i], 0, 0)),
            pl.BlockSpec((1, h, f), lambda i, texp, tval: (texp[i], 0, 0)),
            pl.BlockSpec((1, f, h), lambda i, texp, tval: (texp[i], 0, 0)),
        ],
        out_specs=pl.BlockSpec((TILE, h), lambda i, texp, tval: (i, 0)),
    )
    y = pl.pallas_call(
        _mm_kernel,
        grid_spec=grid_spec,
        out_shape=jax.ShapeDtypeStruct((maxr, h), jnp.float32),
    )(texp, tvalid, xs,
      w_gate.astype(jnp.bfloat16),
      w_up.astype(jnp.bfloat16),
      w_down.astype(jnp.bfloat16))

    combine = pl.kernel(
        functools.partial(_combine_body, tpw=tpw, h=h),
        out_type=jax.ShapeDtypeStruct((t, h), jnp.float32),
        mesh=mesh,
        scratch_types=[
            pltpu.VMEM((tpw,), jnp.int32),
            pltpu.VMEM((tpw,), jnp.int32),
            pltpu.VMEM((tpw,), jnp.float32),
            pltpu.VMEM((tpw,), jnp.float32),
            pltpu.VMEM((L, h), jnp.float32),
            pltpu.VMEM((L, h), jnp.float32),
            pltpu.VMEM((L, h), jnp.float32),
            pltpu.VMEM((L, h), jnp.float32),
            pltpu.SemaphoreType.DMA,
            pltpu.SemaphoreType.DMA,
        ],
    )
    out = combine(pose, poso, w1f, w2f, y)
    return out.reshape(b, s, h)
